# Initial kernel scaffold; baseline (speedup 1.0000x reference)
import jax
import jax.numpy as jnp
from jax.experimental import pallas as pl


def kernel(x, gamma, beta):
    raise NotImplementedError("write your pallas kernel here")


# speedup vs baseline: 1.5450x; 1.5450x over previous
import jax
import jax.numpy as jnp
from jax.experimental import pallas as pl
from jax.experimental.pallas import tpu as pltpu

EPS = 1e-12


def _ln_kernel(x_ref, gamma_ref, beta_ref, o_ref):
    x = x_ref[...]
    y = x + x
    mean = jnp.mean(y, axis=-1, keepdims=True)
    mean_sq = jnp.mean(y * y, axis=-1, keepdims=True)
    var = mean_sq - mean * mean
    yhat = (y - mean) * jax.lax.rsqrt(var + EPS)
    o_ref[...] = yhat * gamma_ref[...] + beta_ref[...]


def kernel(x, gamma, beta):
    b, s, d = x.shape
    n = b * s
    xf = x.reshape(n, d)
    rows = 512
    grid = (n // rows,)
    out = pl.pallas_call(
        _ln_kernel,
        grid=grid,
        in_specs=[
            pl.BlockSpec((rows, d), lambda i: (i, 0)),
            pl.BlockSpec((1, d), lambda i: (0, 0)),
            pl.BlockSpec((1, d), lambda i: (0, 0)),
        ],
        out_specs=pl.BlockSpec((rows, d), lambda i: (i, 0)),
        out_shape=jax.ShapeDtypeStruct((n, d), x.dtype),
        compiler_params=pltpu.CompilerParams(
            dimension_semantics=("parallel",),
        ),
    )(xf, gamma.reshape(1, d), beta.reshape(1, d))
    return out.reshape(b, s, d)


# speedup vs baseline: 1.8424x; 1.1924x over previous
import jax
import jax.numpy as jnp
from jax.experimental import pallas as pl
from jax.experimental.pallas import tpu as pltpu

EPS = 1e-12


def _ln_kernel(x_ref, gamma_ref, beta_ref, o_ref):
    x = x_ref[...]
    y = x + x
    mean = jnp.mean(y, axis=-1, keepdims=True)
    mean_sq = jnp.mean(y * y, axis=-1, keepdims=True)
    var = mean_sq - mean * mean
    yhat = (y - mean) * jax.lax.rsqrt(var + EPS)
    o_ref[...] = yhat * gamma_ref[...] + beta_ref[...]


def kernel(x, gamma, beta):
    b, s, d = x.shape
    n = b * s
    xf = x.reshape(n, d)
    rows = 1024
    grid = (n // rows,)
    out = pl.pallas_call(
        _ln_kernel,
        grid=grid,
        in_specs=[
            pl.BlockSpec((rows, d), lambda i: (i, 0)),
            pl.BlockSpec((1, d), lambda i: (0, 0)),
            pl.BlockSpec((1, d), lambda i: (0, 0)),
        ],
        out_specs=pl.BlockSpec((rows, d), lambda i: (i, 0)),
        out_shape=jax.ShapeDtypeStruct((n, d), x.dtype),
        compiler_params=pltpu.CompilerParams(
            dimension_semantics=("parallel",),
        ),
    )(xf, gamma.reshape(1, d), beta.reshape(1, d))
    return out.reshape(b, s, d)


# speedup vs baseline: 1.8984x; 1.0304x over previous
import jax
import jax.numpy as jnp
from jax.experimental import pallas as pl
from jax.experimental.pallas import tpu as pltpu

EPS = 1e-12


def _ln_kernel(x_ref, gamma_ref, beta_ref, o_ref):
    x = x_ref[...]
    y = x + x
    mean = jnp.mean(y, axis=-1, keepdims=True)
    mean_sq = jnp.mean(y * y, axis=-1, keepdims=True)
    var = mean_sq - mean * mean
    yhat = (y - mean) * jax.lax.rsqrt(var + EPS)
    o_ref[...] = yhat * gamma_ref[...] + beta_ref[...]


def kernel(x, gamma, beta):
    b, s, d = x.shape
    n = b * s
    xf = x.reshape(n, d)
    rows = 2048
    grid = (n // rows,)
    out = pl.pallas_call(
        _ln_kernel,
        grid=grid,
        in_specs=[
            pl.BlockSpec((rows, d), lambda i: (i, 0)),
            pl.BlockSpec((1, d), lambda i: (0, 0)),
            pl.BlockSpec((1, d), lambda i: (0, 0)),
        ],
        out_specs=pl.BlockSpec((rows, d), lambda i: (i, 0)),
        out_shape=jax.ShapeDtypeStruct((n, d), x.dtype),
        compiler_params=pltpu.CompilerParams(
            dimension_semantics=("parallel",),
        ),
    )(xf, gamma.reshape(1, d), beta.reshape(1, d))
    return out.reshape(b, s, d)


# speedup vs baseline: 1.9045x; 1.0032x over previous
import jax
import jax.numpy as jnp
from jax.experimental import pallas as pl
from jax.experimental.pallas import tpu as pltpu

EPS = 1e-12


def _ln_kernel(x_ref, gamma_ref, beta_ref, o_ref):
    x = x_ref[...]
    mean = jnp.mean(x, axis=-1, keepdims=True)
    mean_sq = jnp.mean(x * x, axis=-1, keepdims=True)
    var = mean_sq - mean * mean
    s = jax.lax.rsqrt(var + 0.25 * EPS) * gamma_ref[...]
    t = beta_ref[...] - mean * s
    o_ref[...] = x * s + t


def kernel(x, gamma, beta):
    b, s, d = x.shape
    n = b * s
    xf = x.reshape(n, d)
    rows = 2048
    grid = (n // rows,)
    out = pl.pallas_call(
        _ln_kernel,
        grid=grid,
        in_specs=[
            pl.BlockSpec((rows, d), lambda i: (i, 0)),
            pl.BlockSpec((1, d), lambda i: (0, 0)),
            pl.BlockSpec((1, d), lambda i: (0, 0)),
        ],
        out_specs=pl.BlockSpec((rows, d), lambda i: (i, 0)),
        out_shape=jax.ShapeDtypeStruct((n, d), x.dtype),
        compiler_params=pltpu.CompilerParams(
            dimension_semantics=("parallel",),
        ),
    )(xf, gamma.reshape(1, d), beta.reshape(1, d))
    return out.reshape(b, s, d)
